# baseline (speedup 1.0000x reference)
import functools
import jax
import jax.numpy as jnp
from jax import lax
from jax.experimental import pallas as pl
from jax.experimental.pallas import tpu as pltpu
from jax.experimental.pallas import tpu_sc as plsc

N = 10000
E = 320000
D = 128
CLS = 64
NH = N // 2
NT = 16
EPTD = E // NT
HR = N // NT
C = 640

_mesh = plsc.VectorSubcoreMesh(core_axis_name="c", subcore_axis_name="s")
_sc_params = pltpu.CompilerParams(needs_layout_passes=False)
_sc_params_agg = pltpu.CompilerParams(
    needs_layout_passes=False, use_tc_tiling_on_sc=False
)


CD = 1280
NCD = E // CD


@functools.partial(
    pl.kernel,
    out_type=jax.ShapeDtypeStruct((2, NT, HR, 16), jnp.float32),
    mesh=_mesh,
    compiler_params=_sc_params,
    scratch_types=[
        pltpu.VMEM((CD,), jnp.int32),
        pltpu.VMEM((HR, 16), jnp.float32),
    ],
)
def _deg_kernel(ei_hbm, zeros_hbm, out_hbm, idx_vm, hist_vm):
    c = lax.axis_index("c")
    s = lax.axis_index("s")
    q, rmd = NCD // NT, NCD % NT
    base = s * q + jnp.minimum(s, rmd)
    cnt = q + jnp.where(s < rmd, 1, 0)
    pltpu.sync_copy(zeros_hbm, hist_vm)
    ones = jnp.full((16,), 1.0, jnp.float32)

    def chunk(i, carry):
        pltpu.sync_copy(ei_hbm.at[c].at[pl.ds((base + i) * CD, CD)], idx_vm)

        def step(v, carry2):
            idxv = idx_vm[pl.ds(v * 16, 16)]
            row = lax.shift_right_logical(idxv, 4)
            col = lax.bitwise_and(idxv, 15)
            plsc.addupdate_scatter(hist_vm, [row, col], ones)
            return carry2

        lax.fori_loop(0, CD // 16, step, 0)
        return carry

    lax.fori_loop(0, cnt, chunk, 0)
    pltpu.sync_copy(hist_vm, out_hbm.at[c, s])


def _make_agg(NG, NS):
    EPS = E // NS
    NCH = EPS // C

    @functools.partial(
        pl.kernel,
        out_type=jax.ShapeDtypeStruct((NG, 2, NS, NH, 16), jnp.float32),
        mesh=_mesh,
        compiler_params=_sc_params_agg,
        scratch_types=[
            pltpu.VMEM((NH, 16), jnp.float32),
            pltpu.VMEM((C, 16), jnp.float32),
            pltpu.VMEM((C,), jnp.int32),
            pltpu.VMEM((C,), jnp.int32),
            pltpu.SemaphoreType.DMA,
        ],
    )
    def agg(h_hbm, ei_hbm, zeros_hbm, out_hbm, acc_vm, rows_vm, src_vm, dst_vm, sem):
        c = lax.axis_index("c")
        s = lax.axis_index("s")
        if NS == 2:
            g = lax.shift_right_logical(s, 1)
            dh = lax.bitwise_and(s, 1)
            share = c
        else:
            g = lax.shift_right_logical(s, 2)
            dh = lax.bitwise_and(s, 1)
            share = c * 2 + lax.bitwise_and(lax.shift_right_logical(s, 1), 1)
        lo = dh * NH
        iota = lax.iota(jnp.int32, 16)
        pltpu.sync_copy(zeros_hbm, acc_vm)

        def chunk(jc, carry):
            off = share * EPS + jc * C
            pltpu.sync_copy(ei_hbm.at[0].at[pl.ds(off, C)], src_vm)
            pltpu.sync_copy(ei_hbm.at[1].at[pl.ds(off, C)], dst_vm)
            pltpu.async_copy(h_hbm.at[g].at[src_vm], rows_vm, sem).wait()
            for v in range(C // 16):
                dstv = dst_vm[pl.ds(v * 16, 16)]
                m = jnp.logical_and(dstv >= lo, dstv < lo + NH)
                dstl = dstv - lo
                rowv = iota + (v * 16)
                for j in range(16):
                    col = jnp.full((16,), j, jnp.int32)
                    val = plsc.load_gather(rows_vm, [rowv, col])
                    plsc.addupdate_scatter(acc_vm, [dstl, col], val, mask=m)
            return carry

        lax.fori_loop(0, NCH, chunk, 0)
        pltpu.sync_copy(acc_vm, out_hbm.at[g, dh, share])

    return agg


_agg_l12 = _make_agg(8, 2)
_agg_l3 = _make_agg(4, 4)


R = 1000
NB = N // R


def _norms(degp_ref, which):
    deg = jnp.sum(degp_ref[0, which], axis=0)
    return lax.rsqrt(jnp.maximum(deg, 1.0))


def _tc1_body(degp_ref, x_ref, w_ref, o_ref):
    ns = _norms(degp_ref, 0)
    xn = x_ref[...] * ns[:, None]
    o_ref[0] = jnp.dot(xn, w_ref[0], preferred_element_type=jnp.float32)


_tc1 = pl.pallas_call(
    _tc1_body,
    grid=(NB, 8),
    in_specs=[
        pl.BlockSpec((1, 2, NT, R), lambda i, g: (i, 0, 0, 0)),
        pl.BlockSpec((R, D), lambda i, g: (i, 0)),
        pl.BlockSpec((1, D, 16), lambda i, g: (g, 0, 0)),
    ],
    out_specs=pl.BlockSpec((1, R, 16), lambda i, g: (g, i, 0)),
    out_shape=jax.ShapeDtypeStruct((8, N, 16), jnp.float32),
)


def _mid_body(degp_ref, aggp_ref, b_ref, w_ref, o_ref):
    nd = _norms(degp_ref, 1)
    parts = [jnp.sum(aggp_ref[gi, 0], axis=0) for gi in range(8)]
    agg = jnp.concatenate(parts, axis=1)
    act = jnp.maximum(agg * nd[:, None] + b_ref[0], 0.0)
    ns = _norms(degp_ref, 0)
    xn = act * ns[:, None]
    o_ref[0] = jnp.dot(xn, w_ref[0], preferred_element_type=jnp.float32)


def _make_mid(NG_out):
    return pl.pallas_call(
        _mid_body,
        grid=(NB, NG_out),
        in_specs=[
            pl.BlockSpec((1, 2, NT, R), lambda i, g: (i, 0, 0, 0)),
            pl.BlockSpec((8, 1, 2, R, 16), lambda i, g: (0, i // 5, 0, i % 5, 0)),
            pl.BlockSpec((1, D), lambda i, g: (0, 0)),
            pl.BlockSpec((1, D, 16), lambda i, g: (g, 0, 0)),
        ],
        out_specs=pl.BlockSpec((1, R, 16), lambda i, g: (g, i, 0)),
        out_shape=jax.ShapeDtypeStruct((NG_out, N, 16), jnp.float32),
    )


_mid2 = _make_mid(8)
_mid3 = _make_mid(4)


def _fin_body(degp_ref, aggp_ref, b_ref, o_ref):
    nd = _norms(degp_ref, 1)
    parts = [jnp.sum(aggp_ref[gi, 0], axis=0) for gi in range(4)]
    agg = jnp.concatenate(parts, axis=1)
    o_ref[...] = agg * nd[:, None] + b_ref[0]


_fin = pl.pallas_call(
    _fin_body,
    grid=(NB,),
    in_specs=[
        pl.BlockSpec((1, 2, NT, R), lambda i: (i, 0, 0, 0)),
        pl.BlockSpec((4, 1, 4, R, 16), lambda i: (0, i // 5, 0, i % 5, 0)),
        pl.BlockSpec((1, CLS), lambda i: (0, 0)),
    ],
    out_specs=pl.BlockSpec((R, CLS), lambda i: (i, 0)),
    out_shape=jax.ShapeDtypeStruct((N, CLS), jnp.float32),
)


def kernel(features, edge_index, W1, b1, W2, b2, W3, b3):
    zeros_hr = jnp.zeros((HR, 16), jnp.float32)
    zeros_nh = jnp.zeros((NH, 16), jnp.float32)
    W1g = W1.reshape(D, 8, 16).transpose(1, 0, 2)
    W2g = W2.reshape(D, 8, 16).transpose(1, 0, 2)
    W3g = W3.reshape(D, 4, 16).transpose(1, 0, 2)

    degp = (_deg_kernel(edge_index, zeros_hr)
            .reshape(2, NT, NB, R).transpose(2, 0, 1, 3))

    h1 = _tc1(degp, features, W1g)
    a1 = _agg_l12(h1, edge_index, zeros_nh)
    h2 = _mid2(degp, a1, b1[None, :], W2g)
    a2 = _agg_l12(h2, edge_index, zeros_nh)
    h3 = _mid3(degp, a2, b2[None, :], W3g)
    a3 = _agg_l3(h3, edge_index, zeros_nh)
    return _fin(degp, a3, b3[None, :])


# speedup vs baseline: 10.4616x; 10.4616x over previous
import functools
import jax
import jax.numpy as jnp
from jax import lax
from jax.experimental import pallas as pl
from jax.experimental.pallas import tpu as pltpu
from jax.experimental.pallas import tpu_sc as plsc

N = 10000
E = 320000
D = 128
CLS = 64
NT = 16
HR = N // NT
NRS = N // NT
C = 320

_mesh = plsc.VectorSubcoreMesh(core_axis_name="c", subcore_axis_name="s")
_sc_params = pltpu.CompilerParams(needs_layout_passes=False)
_sc_params_agg = pltpu.CompilerParams(
    needs_layout_passes=False, use_tc_tiling_on_sc=False
)


CD = 1280
NCD = E // CD


@functools.partial(
    pl.kernel,
    out_type=jax.ShapeDtypeStruct((2, NT, HR, 16), jnp.float32),
    mesh=_mesh,
    compiler_params=_sc_params,
    scratch_types=[
        pltpu.VMEM((CD,), jnp.int32),
        pltpu.VMEM((HR, 16), jnp.float32),
    ],
)
def _deg_kernel(ei_hbm, zeros_hbm, out_hbm, idx_vm, hist_vm):
    c = lax.axis_index("c")
    s = lax.axis_index("s")
    q, rmd = NCD // NT, NCD % NT
    base = s * q + jnp.minimum(s, rmd)
    cnt = q + jnp.where(s < rmd, 1, 0)
    pltpu.sync_copy(zeros_hbm, hist_vm)
    ones = jnp.full((16,), 1.0, jnp.float32)

    def chunk(i, carry):
        pltpu.sync_copy(ei_hbm.at[c].at[pl.ds((base + i) * CD, CD)], idx_vm)

        def step(v, carry2):
            idxv = idx_vm[pl.ds(v * 16, 16)]
            row = lax.shift_right_logical(idxv, 4)
            col = lax.bitwise_and(idxv, 15)
            plsc.addupdate_scatter(hist_vm, [row, col], ones)
            return carry2

        lax.fori_loop(0, CD // 16, step, 0)
        return carry

    lax.fori_loop(0, cnt, chunk, 0)
    pltpu.sync_copy(hist_vm, out_hbm.at[c, s])


def _make_agg(DD):
    NCC = (E // C) // 2

    @functools.partial(
        pl.kernel,
        out_type=jax.ShapeDtypeStruct((2, N, DD), jnp.float32),
        mesh=_mesh,
        compiler_params=_sc_params_agg,
        scratch_types=[
            pltpu.VMEM_SHARED((N, DD), jnp.float32),
            pltpu.VMEM((C, DD), jnp.float32),
            pltpu.VMEM((C,), jnp.int32),
            pltpu.VMEM((C,), jnp.int32),
            pltpu.SemaphoreType.DMA,
        ],
    )
    def agg(h_hbm, ei_hbm, zeros_hbm, out_hbm, acc_sh, rows_vm, src_vm, dst_vm, sem):
        c = lax.axis_index("c")
        s = lax.axis_index("s")
        q, rmd = NCC // NT, NCC % NT
        base = c * NCC + s * q + jnp.minimum(s, rmd)
        cnt = q + jnp.where(s < rmd, 1, 0)
        pltpu.sync_copy(
            zeros_hbm.at[pl.ds(s * NRS, NRS)], acc_sh.at[pl.ds(s * NRS, NRS)]
        )
        plsc.subcore_barrier()

        def chunk(i, carry):
            off = (base + i) * C
            pltpu.sync_copy(ei_hbm.at[0].at[pl.ds(off, C)], src_vm)
            pltpu.sync_copy(ei_hbm.at[1].at[pl.ds(off, C)], dst_vm)
            pltpu.async_copy(h_hbm.at[src_vm], rows_vm, sem).wait()
            pltpu.sync_copy(rows_vm, acc_sh.at[dst_vm], add=True)
            return carry

        lax.fori_loop(0, cnt, chunk, 0)
        plsc.subcore_barrier()
        pltpu.sync_copy(
            acc_sh.at[pl.ds(s * NRS, NRS)], out_hbm.at[c].at[pl.ds(s * NRS, NRS)]
        )

    return agg


_agg_d = _make_agg(D)
_agg_c = _make_agg(CLS)


R = 1000
NB = N // R


def _norms(degp_ref, which):
    deg = jnp.sum(degp_ref[0, which], axis=0)
    return lax.rsqrt(jnp.maximum(deg, 1.0))


def _tc1_body(degp_ref, x_ref, w_ref, o_ref):
    ns = _norms(degp_ref, 0)
    xn = x_ref[...] * ns[:, None]
    o_ref[...] = jnp.dot(xn, w_ref[...], preferred_element_type=jnp.float32)


_tc1 = pl.pallas_call(
    _tc1_body,
    grid=(NB,),
    in_specs=[
        pl.BlockSpec((1, 2, NT, R), lambda i: (i, 0, 0, 0)),
        pl.BlockSpec((R, D), lambda i: (i, 0)),
        pl.BlockSpec((D, D), lambda i: (0, 0)),
    ],
    out_specs=pl.BlockSpec((R, D), lambda i: (i, 0)),
    out_shape=jax.ShapeDtypeStruct((N, D), jnp.float32),
)


def _mid_body(degp_ref, aggp_ref, b_ref, w_ref, o_ref):
    nd = _norms(degp_ref, 1)
    agg = aggp_ref[0] + aggp_ref[1]
    act = jnp.maximum(agg * nd[:, None] + b_ref[0], 0.0)
    ns = _norms(degp_ref, 0)
    xn = act * ns[:, None]
    o_ref[...] = jnp.dot(xn, w_ref[...], preferred_element_type=jnp.float32)


def _make_mid(DO):
    return pl.pallas_call(
        _mid_body,
        grid=(NB,),
        in_specs=[
            pl.BlockSpec((1, 2, NT, R), lambda i: (i, 0, 0, 0)),
            pl.BlockSpec((2, R, D), lambda i: (0, i, 0)),
            pl.BlockSpec((1, D), lambda i: (0, 0)),
            pl.BlockSpec((D, DO), lambda i: (0, 0)),
        ],
        out_specs=pl.BlockSpec((R, DO), lambda i: (i, 0)),
        out_shape=jax.ShapeDtypeStruct((N, DO), jnp.float32),
    )


_mid2 = _make_mid(D)
_mid3 = _make_mid(CLS)


def _fin_body(degp_ref, aggp_ref, b_ref, o_ref):
    nd = _norms(degp_ref, 1)
    agg = aggp_ref[0] + aggp_ref[1]
    o_ref[...] = agg * nd[:, None] + b_ref[0]


_fin = pl.pallas_call(
    _fin_body,
    grid=(NB,),
    in_specs=[
        pl.BlockSpec((1, 2, NT, R), lambda i: (i, 0, 0, 0)),
        pl.BlockSpec((2, R, CLS), lambda i: (0, i, 0)),
        pl.BlockSpec((1, CLS), lambda i: (0, 0)),
    ],
    out_specs=pl.BlockSpec((R, CLS), lambda i: (i, 0)),
    out_shape=jax.ShapeDtypeStruct((N, CLS), jnp.float32),
)


def kernel(features, edge_index, W1, b1, W2, b2, W3, b3):
    zeros_hr = jnp.zeros((HR, 16), jnp.float32)
    zeros_nd = jnp.zeros((N, D), jnp.float32)
    zeros_nc = jnp.zeros((N, CLS), jnp.float32)

    degp = (_deg_kernel(edge_index, zeros_hr)
            .reshape(2, NT, NB, R).transpose(2, 0, 1, 3))

    h1 = _tc1(degp, features, W1)
    a1 = _agg_d(h1, edge_index, zeros_nd)
    h2 = _mid2(degp, a1, b1[None, :], W2)
    a2 = _agg_d(h2, edge_index, zeros_nd)
    h3 = _mid3(degp, a2, b2[None, :], W3)
    a3 = _agg_c(h3, edge_index, zeros_nc)
    return _fin(degp, a3, b3[None, :])
